# baseline (speedup 1.0000x reference)
import functools
import math

import jax
import jax.numpy as jnp
from jax import lax
from jax.experimental import pallas as pl
from jax.experimental.pallas import tpu as pltpu
from jax.experimental.pallas import tpu_sc as plsc


@functools.lru_cache(maxsize=None)
def _build(B, T, D, V):
    info = plsc.get_sparse_core_info()
    NC, NS, L = info.num_cores, info.num_subcores, info.num_lanes
    NW = NC * NS
    BT = 128
    TG = 8
    assert B % (NW * BT) == 0 and T % TG == 0 and D % L == 0
    NBT = B // BT
    cpw = NBT // NW
    ntg = T // TG
    scale = float(math.sqrt(D))

    mesh = plsc.VectorSubcoreMesh(core_axis_name="c", subcore_axis_name="s")

    @functools.partial(
        pl.kernel,
        out_type=jax.ShapeDtypeStruct((T, D // 8, NBT, 8, BT), jnp.float32),
        mesh=mesh,
        compiler_params=pltpu.CompilerParams(use_tc_tiling_on_sc=False,
                                             needs_layout_passes=False),
        scratch_types=[
            pltpu.VMEM((T, D), jnp.float32),
            pltpu.VMEM((2, TG, BT), jnp.int32),
            pltpu.VMEM((TG, BT, D), jnp.float32),
            pltpu.VMEM((BT, D), jnp.float32),
            pltpu.VMEM((2, D // 8, 8, BT), jnp.float32),
            pltpu.SemaphoreType.DMA,
            pltpu.SemaphoreType.DMA,
            pltpu.SemaphoreType.DMA,
        ],
    )
    def launch(x4, tok_hbm, pos_hbm, o5, posblk, xc, rowb, rowp, outt,
               xsem, gsem, osem):
        wid = lax.axis_index("s") * NC + lax.axis_index("c")
        c0 = wid * cpw
        pltpu.sync_copy(pos_hbm.at[pl.ds(0, T)], posblk)
        lanes = lax.iota(jnp.int32, L)

        for cl in range(cpw):
            c = c0 + cl
            pltpu.sync_copy(x4.at[0, c], xc.at[0])
            for tt in range(TG):
                pltpu.async_copy(tok_hbm.at[xc.at[0, tt]], rowb.at[tt],
                                 gsem)

            def tgbody(tg, carry, cl=cl, c=c):
                @pl.when(tg + 1 < ntg)
                def _():
                    pltpu.async_copy(x4.at[tg + 1, c],
                                     xc.at[(tg + 1) % 2], xsem)

                def ttbody(tt, carry2, cl=cl, c=c):
                    t = tg * TG + tt
                    par = tt % 2
                    pltpu.make_async_copy(
                        tok_hbm.at[pl.ds(0, BT)], rowb.at[0], gsem).wait()

                    pv = [posblk[t, pl.ds(k * L, L)] for k in range(D // L)]

                    def p1body(b, c3):
                        rot = (lanes - b) & (L - 1)
                        for k in range(D // L):
                            v = rowb[tt, b, pl.ds(k * L, L)]
                            v = v * scale + pv[k]
                            v = v.at[rot].get(mode="promise_in_bounds")
                            rowp[b, pl.ds(k * L, L)] = v
                        return c3

                    lax.fori_loop(0, BT, p1body, 0, unroll=4)

                    @pl.when(tg + 1 < ntg)
                    def _():
                        @pl.when(tt == 0)
                        def _():
                            pltpu.make_async_copy(
                                x4.at[0, c], xc.at[0], xsem).wait()
                        pltpu.async_copy(
                            tok_hbm.at[xc.at[(tg + 1) % 2, tt]],
                            rowb.at[tt], gsem)

                    if cl > 0:
                        pltpu.make_async_copy(
                            outt.at[0], o5.at[0, :, 0], osem).wait()
                    else:
                        @pl.when(tg * TG + tt >= 2)
                        def _():
                            pltpu.make_async_copy(
                                outt.at[0], o5.at[0, :, 0], osem).wait()

                    def p2body(d, c3):
                        j = d // 8
                        dd = d % 8
                        col = ((d % L) + lanes) & (L - 1)
                        cols = (d // L) * L + col
                        for m in range(BT // L):
                            bs = lanes + (m * L)
                            v = plsc.load_gather(rowp, [bs, cols])
                            outt[par, j, dd, pl.ds(m * L, L)] = v
                        return c3

                    lax.fori_loop(0, D, p2body, 0, unroll=4)
                    pltpu.async_copy(outt.at[par], o5.at[t, :, c], osem)
                    return carry2

                lax.fori_loop(0, TG, ttbody, 0)
                return carry

            lax.fori_loop(0, ntg, tgbody, 0)

        for _ in range(2):
            pltpu.make_async_copy(outt.at[0], o5.at[0, :, 0], osem).wait()

    return launch


def kernel(x, token_table, pos_table):
    B, T = x.shape
    V, D = token_table.shape
    launch = _build(B, T, D, V)
    x4 = (x.astype(jnp.int32).T
          .reshape(T // 8, 8, B // 128, 128)
          .transpose(0, 2, 1, 3))
    o5 = launch(x4, token_table, pos_table)
    return o5.transpose(2, 4, 0, 1, 3).reshape(B, T, D)


# speedup vs baseline: 4.0390x; 4.0390x over previous
import functools
import math

import jax
import jax.numpy as jnp
from jax import lax
from jax.experimental import pallas as pl
from jax.experimental.pallas import tpu as pltpu
from jax.experimental.pallas import tpu_sc as plsc


@functools.lru_cache(maxsize=None)
def _build(B, T, D, V):
    info = plsc.get_sparse_core_info()
    NC, NS, L = info.num_cores, info.num_subcores, info.num_lanes
    NW = NC * NS
    BT = 128
    TG = 8
    assert B % (NW * BT) == 0 and T % TG == 0 and D % L == 0
    NBT = B // BT
    cpw = NBT // NW
    ntg = T // TG
    scale = float(math.sqrt(D))

    mesh = plsc.VectorSubcoreMesh(core_axis_name="c", subcore_axis_name="s")

    @functools.partial(
        pl.kernel,
        out_type=jax.ShapeDtypeStruct((T, D // 8, NBT, 8, BT), jnp.float32),
        mesh=mesh,
        compiler_params=pltpu.CompilerParams(use_tc_tiling_on_sc=False,
                                             needs_layout_passes=False),
        scratch_types=[
            pltpu.VMEM((T, D), jnp.float32),
            pltpu.VMEM((2, TG, BT), jnp.int32),
            pltpu.VMEM((TG, BT, D), jnp.float32),
            pltpu.VMEM((BT, D), jnp.float32),
            pltpu.VMEM((2, D // 8, 8, BT), jnp.float32),
            pltpu.SemaphoreType.DMA,
            pltpu.SemaphoreType.DMA,
            pltpu.SemaphoreType.DMA,
        ],
    )
    def launch(x4, tok_hbm, pos_hbm, o5, posblk, xc, rowb, rowp, outt,
               xsem, gsem, osem):
        wid = lax.axis_index("s") * NC + lax.axis_index("c")
        c0 = wid * cpw
        pltpu.sync_copy(pos_hbm.at[pl.ds(0, T)], posblk)
        lanes = lax.iota(jnp.int32, L)

        for cl in range(cpw):
            c = c0 + cl
            pltpu.sync_copy(x4.at[0, c], xc.at[0])
            for tt in range(TG):
                pltpu.async_copy(tok_hbm.at[xc.at[0, tt]], rowb.at[tt],
                                 gsem)

            def tgbody(tg, carry, cl=cl, c=c):
                @pl.when(tg + 1 < ntg)
                def _():
                    pltpu.async_copy(x4.at[tg + 1, c],
                                     xc.at[(tg + 1) % 2], xsem)

                def ttbody(tt, carry2, cl=cl, c=c):
                    t = tg * TG + tt
                    par = tt % 2
                    pltpu.make_async_copy(
                        tok_hbm.at[pl.ds(0, BT)], rowb.at[0], gsem).wait()

                    pv = [posblk[t, pl.ds(k * L, L)] for k in range(D // L)]

                    @plsc.parallel_loop(0, BT, unroll=4)
                    def _(b, tt=tt, pv=pv):
                        rot = (lanes - b) & (L - 1)
                        for k in range(D // L):
                            v = rowb[tt, b, pl.ds(k * L, L)]
                            v = v * scale + pv[k]
                            v = v.at[rot].get(mode="promise_in_bounds")
                            rowp[b, pl.ds(k * L, L)] = v

                    @pl.when(tg + 1 < ntg)
                    def _():
                        @pl.when(tt == 0)
                        def _():
                            pltpu.make_async_copy(
                                x4.at[0, c], xc.at[0], xsem).wait()
                        pltpu.async_copy(
                            tok_hbm.at[xc.at[(tg + 1) % 2, tt]],
                            rowb.at[tt], gsem)

                    if cl > 0:
                        pltpu.make_async_copy(
                            outt.at[0], o5.at[0, :, 0], osem).wait()
                    else:
                        @pl.when(tg * TG + tt >= 2)
                        def _():
                            pltpu.make_async_copy(
                                outt.at[0], o5.at[0, :, 0], osem).wait()

                    @plsc.parallel_loop(0, D, unroll=4)
                    def _(d, par=par):
                        j = d // 8
                        dd = d % 8
                        col = ((d % L) + lanes) & (L - 1)
                        cols = (d // L) * L + col
                        for m in range(BT // L):
                            bs = lanes + (m * L)
                            v = plsc.load_gather(rowp, [bs, cols])
                            outt[par, j, dd, pl.ds(m * L, L)] = v
                    pltpu.async_copy(outt.at[par], o5.at[t, :, c], osem)
                    return carry2

                lax.fori_loop(0, TG, ttbody, 0)
                return carry

            lax.fori_loop(0, ntg, tgbody, 0)

        for _ in range(2):
            pltpu.make_async_copy(outt.at[0], o5.at[0, :, 0], osem).wait()

    return launch


def kernel(x, token_table, pos_table):
    B, T = x.shape
    V, D = token_table.shape
    launch = _build(B, T, D, V)
    x4 = (x.astype(jnp.int32).T
          .reshape(T // 8, 8, B // 128, 128)
          .transpose(0, 2, 1, 3))
    o5 = launch(x4, token_table, pos_table)
    return o5.transpose(2, 4, 0, 1, 3).reshape(B, T, D)


# speedup vs baseline: 4.0607x; 1.0054x over previous
import functools
import math

import jax
import jax.numpy as jnp
from jax import lax
from jax.experimental import pallas as pl
from jax.experimental.pallas import tpu as pltpu
from jax.experimental.pallas import tpu_sc as plsc


@functools.lru_cache(maxsize=None)
def _build(B, T, D, V):
    info = plsc.get_sparse_core_info()
    NC, NS, L = info.num_cores, info.num_subcores, info.num_lanes
    NW = NC * NS
    BT = 128
    TG = 8
    assert B % (NW * BT) == 0 and T % TG == 0 and D % L == 0
    NBT = B // BT
    cpw = NBT // NW
    ntg = T // TG
    scale = float(math.sqrt(D))

    mesh = plsc.VectorSubcoreMesh(core_axis_name="c", subcore_axis_name="s")

    @functools.partial(
        pl.kernel,
        out_type=jax.ShapeDtypeStruct((T, D // 8, NBT, 8, BT), jnp.float32),
        mesh=mesh,
        compiler_params=pltpu.CompilerParams(use_tc_tiling_on_sc=False,
                                             needs_layout_passes=False),
        scratch_types=[
            pltpu.VMEM((T, D), jnp.float32),
            pltpu.VMEM((2, TG, BT), jnp.int32),
            pltpu.VMEM((TG, BT, D), jnp.float32),
            pltpu.VMEM((BT, D), jnp.float32),
            pltpu.VMEM((2, D // 8, 8, BT), jnp.float32),
            pltpu.SemaphoreType.DMA,
            pltpu.SemaphoreType.DMA,
            pltpu.SemaphoreType.DMA,
        ],
    )
    def launch(x4, tok_hbm, pos_hbm, o5, posblk, xc, rowb, rowp, outt,
               xsem, gsem, osem):
        wid = lax.axis_index("s") * NC + lax.axis_index("c")
        c0 = wid * cpw
        pltpu.sync_copy(pos_hbm.at[pl.ds(0, T)], posblk)
        lanes = lax.iota(jnp.int32, L)

        for cl in range(cpw):
            c = c0 + cl
            pltpu.sync_copy(x4.at[0, c], xc.at[0])
            for tt in range(TG):
                pltpu.async_copy(tok_hbm.at[xc.at[0, tt]], rowb.at[tt],
                                 gsem)

            def tgbody(tg, carry, cl=cl, c=c):
                @pl.when(tg + 1 < ntg)
                def _():
                    pltpu.async_copy(x4.at[tg + 1, c],
                                     xc.at[(tg + 1) % 2], xsem)

                def ttbody(tt, carry2, cl=cl, c=c):
                    t = tg * TG + tt
                    par = tt % 2
                    pltpu.make_async_copy(
                        tok_hbm.at[pl.ds(0, BT)], rowb.at[0], gsem).wait()

                    pv = [posblk[t, pl.ds(k * L, L)] for k in range(D // L)]

                    @plsc.parallel_loop(0, BT, unroll=8)
                    def _(b, tt=tt, pv=pv):
                        rot = (lanes - b) & (L - 1)
                        for k in range(D // L):
                            v = rowb[tt, b, pl.ds(k * L, L)]
                            v = v * scale + pv[k]
                            v = v.at[rot].get(mode="promise_in_bounds")
                            rowp[b, pl.ds(k * L, L)] = v

                    @pl.when(tg + 1 < ntg)
                    def _():
                        @pl.when(tt == 0)
                        def _():
                            pltpu.make_async_copy(
                                x4.at[0, c], xc.at[0], xsem).wait()
                        pltpu.async_copy(
                            tok_hbm.at[xc.at[(tg + 1) % 2, tt]],
                            rowb.at[tt], gsem)

                    if cl > 0:
                        pltpu.make_async_copy(
                            outt.at[0], o5.at[0, :, 0], osem).wait()
                    else:
                        @pl.when(tg * TG + tt >= 2)
                        def _():
                            pltpu.make_async_copy(
                                outt.at[0], o5.at[0, :, 0], osem).wait()

                    @plsc.parallel_loop(0, D, unroll=8)
                    def _(d, par=par):
                        j = d // 8
                        dd = d % 8
                        col = ((d % L) + lanes) & (L - 1)
                        cols = (d // L) * L + col
                        for m in range(BT // L):
                            bs = lanes + (m * L)
                            v = plsc.load_gather(rowp, [bs, cols])
                            outt[par, j, dd, pl.ds(m * L, L)] = v
                    pltpu.async_copy(outt.at[par], o5.at[t, :, c], osem)
                    return carry2

                lax.fori_loop(0, TG, ttbody, 0)
                return carry

            lax.fori_loop(0, ntg, tgbody, 0)

        for _ in range(2):
            pltpu.make_async_copy(outt.at[0], o5.at[0, :, 0], osem).wait()

    return launch


def kernel(x, token_table, pos_table):
    B, T = x.shape
    V, D = token_table.shape
    launch = _build(B, T, D, V)
    x4 = (x.astype(jnp.int32).T
          .reshape(T // 8, 8, B // 128, 128)
          .transpose(0, 2, 1, 3))
    o5 = launch(x4, token_table, pos_table)
    return o5.transpose(2, 4, 0, 1, 3).reshape(B, T, D)
